# baseline (speedup 1.0000x reference)
import dataclasses

import jax
import jax.numpy as jnp
from jax import lax
from jax.experimental import pallas as pl
from jax.experimental.pallas import tpu as pltpu
from jax.experimental.pallas import tpu_sc as plsc

NUM_FIELDS = 26
VOCAB_SIZE = 100000
K = 16
BATCH = 16384
LANES = 16

NUM_CORES = 2
NUM_SUBCORES = 16
NW = NUM_CORES * NUM_SUBCORES
B_PER_W = BATCH // NW
CB = 128
NCHUNK = B_PER_W // CB
GROUPS = CB // LANES


def _fm_body(idx_hbm, w_hbm, v_hbm, out_hbm,
             idx_v, wvals_v, vrows_v, dbuf_v, outbuf_v, sem):
    wid = lax.axis_index("c") * NUM_SUBCORES + lax.axis_index("s")
    base = wid * B_PER_W

    @pl.loop(0, NCHUNK)
    def _chunk(ci):
        col0 = base + ci * CB
        pltpu.sync_copy(idx_hbm.at[:, pl.ds(col0, CB)], idx_v)

        handles = []
        for f in range(NUM_FIELDS):
            handles.append(pltpu.async_copy(
                v_hbm.at[f].at[idx_v.at[f]],
                vrows_v.at[pl.ds(f * CB, CB), :], sem))
            handles.append(pltpu.async_copy(
                w_hbm.at[f].at[idx_v.at[f]],
                wvals_v.at[f], sem))
        for h in handles:
            h.wait()

        @pl.loop(0, GROUPS)
        def _group(g):
            b0 = g * LANES
            acc_w = jnp.zeros((LANES,), jnp.float32)
            for f in range(NUM_FIELDS):
                acc_w = acc_w + wvals_v[f, pl.ds(b0, LANES)]

            @pl.loop(0, LANES)
            def _b(bl):
                b = b0 + bl
                acc_s = jnp.zeros((LANES,), jnp.float32)
                acc_q = jnp.zeros((LANES,), jnp.float32)
                for f in range(NUM_FIELDS):
                    row = vrows_v[f * CB + b, :]
                    acc_s = acc_s + row
                    acc_q = acc_q + row * row
                dbuf_v[pl.ds(bl * LANES, LANES)] = acc_s * acc_s - acc_q

            lanes_iota = lax.iota(jnp.int32, LANES)
            tsum = jnp.zeros((LANES,), jnp.float32)
            for j in range(LANES):
                tsum = tsum + plsc.load_gather(
                    dbuf_v, [lanes_iota * LANES + j])
            outbuf_v[pl.ds(b0, LANES)] = acc_w + 0.5 * tsum

        pltpu.sync_copy(outbuf_v, out_hbm.at[pl.ds(col0, CB)])


def kernel(indices, w_tables, v_tables):
    idx_t = indices.T.astype(jnp.int32)
    mesh = plsc.VectorSubcoreMesh(core_axis_name="c", subcore_axis_name="s")
    cp = pltpu.CompilerParams()
    if "needs_layout_passes" in pltpu.CompilerParams.__dataclass_fields__:
        cp = dataclasses.replace(cp, needs_layout_passes=False)
    if "use_tc_tiling_on_sc" in pltpu.CompilerParams.__dataclass_fields__:
        cp = dataclasses.replace(cp, use_tc_tiling_on_sc=False)
    fm = pl.kernel(
        _fm_body,
        out_type=jax.ShapeDtypeStruct((BATCH,), jnp.float32),
        mesh=mesh,
        scratch_types=[
            pltpu.VMEM((NUM_FIELDS, CB), jnp.int32),
            pltpu.VMEM((NUM_FIELDS, CB), jnp.float32),
            pltpu.VMEM((NUM_FIELDS * CB, K), jnp.float32),
            pltpu.VMEM((LANES * LANES,), jnp.float32),
            pltpu.VMEM((CB,), jnp.float32),
            pltpu.SemaphoreType.DMA,
        ],
        compiler_params=cp,
    )
    return fm(idx_t, w_tables, v_tables)
